# baseline (speedup 1.0000x reference)
import functools

import jax
import jax.numpy as jnp
import numpy as np
from jax import lax
from jax.experimental import pallas as pl
from jax.experimental.pallas import tpu as pltpu
from jax.experimental.pallas import tpu_sc as plsc

N = 10000
E = 320000
HID = 128
OUT = 40
CHUNK = 128
NW = 32
NCK = 80
ROWS = NW * NCK
E_PAD = ROWS * CHUNK
NACC = 10240
ZROWS = NACC // 16
BN = 400
GRID = N // BN

_f32 = jnp.float32


NCH = ZROWS // CHUNK


def _fill_stripe(stage, shared, sid):
    def body(t, carry):
        pltpu.sync_copy(stage,
                        shared.at[pl.ds(sid * ZROWS + t * CHUNK, CHUNK)])
        return carry
    lax.fori_loop(0, NCH, body, 0)


def _copy_out(shared, out_ref, stage, cid, sid):
    def body(t, carry):
        off = sid * ZROWS + t * CHUNK
        pltpu.sync_copy(shared.at[pl.ds(off, CHUNK)], stage)
        pltpu.sync_copy(stage, out_ref.at[cid, pl.ds(off, CHUNK)])
        return carry
    lax.fori_loop(0, NCH, body, 0)


def _sc_mesh():
    return plsc.VectorSubcoreMesh(core_axis_name="c", subcore_axis_name="s",
                                  num_cores=2, num_subcores=16)


def _count_kernel(dst_hbm, z128_hbm, o128_hbm, cnt_out, da, db, buf, cacc,
                  sema, semb):
    cid = lax.axis_index("c")
    sid = lax.axis_index("s")
    wid = sid * 2 + cid
    base = wid * NCK * CHUNK
    pltpu.sync_copy(z128_hbm, buf)
    _fill_stripe(buf, cacc, sid)
    pltpu.sync_copy(o128_hbm, buf)
    plsc.subcore_barrier()

    pltpu.async_copy(dst_hbm.at[pl.ds(base, CHUNK)], da, sema)
    pltpu.async_copy(dst_hbm.at[pl.ds(base + CHUNK, CHUNK)], db, semb)

    def body(kk, carry):
        k0 = 2 * kk
        pltpu.make_async_copy(dst_hbm.at[pl.ds(base, CHUNK)], da,
                              sema).wait()
        pltpu.sync_copy(buf, cacc.at[da], add=True)

        @pl.when(k0 + 2 < NCK)
        def _():
            off = base + (k0 + 2) * CHUNK
            pltpu.async_copy(dst_hbm.at[pl.ds(off, CHUNK)], da, sema)

        pltpu.make_async_copy(dst_hbm.at[pl.ds(base, CHUNK)], db,
                              semb).wait()
        pltpu.sync_copy(buf, cacc.at[db], add=True)

        @pl.when(k0 + 3 < NCK)
        def _():
            off = base + (k0 + 3) * CHUNK
            pltpu.async_copy(dst_hbm.at[pl.ds(off, CHUNK)], db, semb)

        return carry

    lax.fori_loop(0, NCK // 2, body, 0)
    plsc.subcore_barrier()
    _copy_out(cacc, cnt_out, buf, cid, sid)


def _agg_kernel(src_hbm, dst_hbm, table_hbm, z128_hbm, acc_out,
                sa, da, rows, acc, sg):
    cid = lax.axis_index("c")
    sid = lax.axis_index("s")
    wid = sid * 2 + cid
    pltpu.sync_copy(z128_hbm, rows)
    _fill_stripe(rows, acc, sid)
    plsc.subcore_barrier()

    def body(k, carry):
        off = (wid + k * NW) * CHUNK
        pltpu.sync_copy(src_hbm.at[pl.ds(off, CHUNK)], sa)
        pltpu.sync_copy(dst_hbm.at[pl.ds(off, CHUNK)], da)
        pltpu.async_copy(table_hbm.at[sa], rows, sg).wait()
        pltpu.sync_copy(rows, acc.at[da], add=True)
        return carry

    lax.fori_loop(0, NCK, body, 0)
    plsc.subcore_barrier()
    _copy_out(acc, acc_out, rows, cid, sid)


def _sc_counts(dst1d, z128, o128):
    fn = pl.kernel(
        _count_kernel,
        out_type=jax.ShapeDtypeStruct((2, NACC, HID), _f32),
        mesh=_sc_mesh(),
        scratch_types=[
            pltpu.VMEM((CHUNK,), jnp.int32),
            pltpu.VMEM((CHUNK,), jnp.int32),
            pltpu.VMEM((CHUNK, HID), _f32),
            pltpu.VMEM_SHARED((NACC, HID), _f32),
            pltpu.SemaphoreType.DMA,
            pltpu.SemaphoreType.DMA,
        ],
    )
    return fn(dst1d, z128, o128)


def _sc_aggregate(src1d, dst1d, table, z128):
    fn = pl.kernel(
        _agg_kernel,
        out_type=jax.ShapeDtypeStruct((2, NACC, HID), _f32),
        mesh=_sc_mesh(),
        scratch_types=[
            pltpu.VMEM((CHUNK,), jnp.int32),
            pltpu.VMEM((CHUNK,), jnp.int32),
            pltpu.VMEM((CHUNK, HID), _f32),
            pltpu.VMEM_SHARED((NACC, HID), _f32),
            pltpu.SemaphoreType.DMA,
        ],
    )
    return fn(src1d, dst1d, table, z128)


def _frontend_body(xf, hist, K, bfull, wsa, wsb, lsb, wna, wnb, lnb,
                   selfx, xn):
    u = jnp.maximum(xf[...] @ K[...] + bfull[...], 0.0)
    hb = jnp.maximum(hist[...], 0.0)
    selfx[...] = u @ wsa[...] + hb @ wsb[...] + lsb[...]
    xn[...] = u @ wna[...] + hb @ wnb[...] + lnb[...]


def _frontend(xf, hist, K, bfull, wsa, wsb, lsb, wna, wnb, lnb):
    row = lambda i: (i, 0)
    full = lambda i: (0, 0)
    return pl.pallas_call(
        _frontend_body,
        grid=(GRID,),
        in_specs=[
            pl.BlockSpec((BN, 400), row),
            pl.BlockSpec((BN, 16), row),
            pl.BlockSpec((400, 256), full),
            pl.BlockSpec((1, 256), full),
            pl.BlockSpec((256, HID), full),
            pl.BlockSpec((16, HID), full),
            pl.BlockSpec((1, HID), full),
            pl.BlockSpec((256, HID), full),
            pl.BlockSpec((16, HID), full),
            pl.BlockSpec((1, HID), full),
        ],
        out_specs=[pl.BlockSpec((BN, HID), row), pl.BlockSpec((BN, HID), row)],
        out_shape=[jax.ShapeDtypeStruct((N, HID), _f32),
                   jax.ShapeDtypeStruct((N, HID), _f32)],
    )(xf, hist, K, bfull, wsa, wsb, lsb, wna, wnb, lnb)


def _combine1_body(selfx, a0, a1, h0):
    h0[...] = selfx[...] + a0[...] + a1[...]


def _combine1(selfx, a0, a1):
    row = lambda i: (i, 0)
    return pl.pallas_call(
        _combine1_body,
        grid=(GRID,),
        in_specs=[pl.BlockSpec((BN, HID), row)] * 3,
        out_specs=pl.BlockSpec((BN, HID), row),
        out_shape=jax.ShapeDtypeStruct((N, HID), _f32),
    )(selfx, a0, a1)


def _sage_ln_body(a0, a1, c0, c1, h, lwt, lb, rwt, g, b, out):
    cnt = jnp.maximum(c0[...] + c1[...], 1.0)
    mean = (a0[...] + a1[...]) / cnt
    s = mean @ lwt[...] + lb[...] + h[...] @ rwt[...]
    r = jnp.maximum(s, 0.0)
    m = jnp.mean(r, axis=-1, keepdims=True)
    v = jnp.mean((r - m) ** 2, axis=-1, keepdims=True)
    out[...] = (r - m) / jnp.sqrt(v + 1e-5) * g[...] + b[...]


def _sage_ln(a0, a1, c0, c1, h, lwt, lb, rwt, g, b):
    row = lambda i: (i, 0)
    rowc = lambda i: (i, 0)
    full = lambda i: (0, 0)
    return pl.pallas_call(
        _sage_ln_body,
        grid=(GRID,),
        in_specs=[
            pl.BlockSpec((BN, HID), row),
            pl.BlockSpec((BN, HID), row),
            pl.BlockSpec((BN, 1), rowc),
            pl.BlockSpec((BN, 1), rowc),
            pl.BlockSpec((BN, HID), row),
            pl.BlockSpec((HID, HID), full),
            pl.BlockSpec((1, HID), full),
            pl.BlockSpec((HID, HID), full),
            pl.BlockSpec((1, HID), full),
            pl.BlockSpec((1, HID), full),
        ],
        out_specs=pl.BlockSpec((BN, HID), row),
        out_shape=jax.ShapeDtypeStruct((N, HID), _f32),
    )(a0, a1, c0, c1, h, lwt, lb, rwt, g, b)


def _final_body(a0, a1, c0, c1, h, lwt, lb, rwt, p1t, p1b, p2t, p2b,
                emb, logp):
    cnt = jnp.maximum(c0[...] + c1[...], 1.0)
    mean = (a0[...] + a1[...]) / cnt
    s = mean @ lwt[...] + lb[...] + h[...] @ rwt[...]
    emb[...] = s
    r = jnp.maximum(s, 0.0)
    o = r @ p1t[...] + p1b[...]
    o2 = o @ p2t[...] + p2b[...]
    mx = jnp.max(o2, axis=-1, keepdims=True)
    ex = jnp.exp(o2 - mx)
    lse = jnp.log(jnp.sum(ex, axis=-1, keepdims=True)) + mx
    logp[...] = o2 - lse


def _final(a0, a1, c0, c1, h, lwt, lb, rwt, p1t, p1b, p2t, p2b):
    row = lambda i: (i, 0)
    full = lambda i: (0, 0)
    return pl.pallas_call(
        _final_body,
        grid=(GRID,),
        in_specs=[
            pl.BlockSpec((BN, HID), row),
            pl.BlockSpec((BN, HID), row),
            pl.BlockSpec((BN, 1), row),
            pl.BlockSpec((BN, 1), row),
            pl.BlockSpec((BN, HID), row),
            pl.BlockSpec((HID, HID), full),
            pl.BlockSpec((1, HID), full),
            pl.BlockSpec((HID, HID), full),
            pl.BlockSpec((HID, HID), full),
            pl.BlockSpec((1, HID), full),
            pl.BlockSpec((HID, OUT), full),
            pl.BlockSpec((1, OUT), full),
        ],
        out_specs=[pl.BlockSpec((BN, HID), row), pl.BlockSpec((BN, OUT), row)],
        out_shape=[jax.ShapeDtypeStruct((N, HID), _f32),
                   jax.ShapeDtypeStruct((N, OUT), _f32)],
    )(a0, a1, c0, c1, h, lwt, lb, rwt, p1t, p1b, p2t, p2b)


_o, _c, _di, _dj, _i, _j = np.meshgrid(
    np.arange(4), np.arange(4), np.arange(3), np.arange(3),
    np.arange(8), np.arange(8), indexing="ij")
_RIN = (_c * 100 + (_i + _di) * 10 + (_j + _dj)).reshape(-1)
_ROUT = (_o * 64 + _i * 8 + _j).reshape(-1)


def _conv_as_matmul(conv_W):
    vals = jnp.broadcast_to(conv_W[:, :, :, :, None, None],
                            (4, 4, 3, 3, 8, 8)).reshape(-1)
    return jnp.zeros((400, 256), _f32).at[_RIN, _ROUT].set(vals)


def kernel(x, histories_preds, edge_index, batch, conv_self_W, conv_self_b,
           lin_self_W, lin_self_b, lin_W, lin_b,
           sage1_lW, sage1_lb, sage1_rW, sage2_lW, sage2_lb, sage2_rW,
           sage3_lW, sage3_lb, sage3_rW,
           ln1_g, ln1_b, ln2_g, ln2_b, post1_W, post1_b, post2_W, post2_b):
    src = edge_index[0]
    dst = edge_index[1]
    dstm = jnp.where(src == dst, N, dst)
    pad = E_PAD - E
    srcp = jnp.concatenate([src, jnp.zeros((pad,), jnp.int32)])
    dstp = jnp.concatenate([dst, jnp.full((pad,), N, jnp.int32)])
    dstmp = jnp.concatenate([dstm, jnp.full((pad,), N, jnp.int32)])

    xf = x.reshape(N, 400)
    K = _conv_as_matmul(conv_self_W)
    bfull = jnp.repeat(conv_self_b, 64).reshape(1, 256)
    wsa = lin_self_W[:, :256].T
    wsb = lin_self_W[:, 256:].T
    wna = lin_W[:, :256].T
    wnb = lin_W[:, 256:].T

    z128 = jnp.zeros((CHUNK, HID), _f32)
    o128 = jnp.ones((CHUNK, HID), _f32)

    selfx, xn = _frontend(xf, histories_preds, K, bfull,
                          wsa, wsb, lin_self_b.reshape(1, HID),
                          wna, wnb, lin_b.reshape(1, HID))

    cnt = _sc_counts(dstp, z128, o128)
    acc1 = _sc_aggregate(srcp, dstmp, xn, z128)
    c0 = cnt[0, :N, 0].reshape(N, 1)
    c1 = cnt[1, :N, 0].reshape(N, 1)
    h0 = _combine1(selfx, acc1[0, :N], acc1[1, :N])

    agg = _sc_aggregate(srcp, dstp, h0, z128)
    h1 = _sage_ln(agg[0, :N], agg[1, :N], c0, c1, h0,
                  sage1_lW.T, sage1_lb.reshape(1, HID), sage1_rW.T,
                  ln1_g.reshape(1, HID), ln1_b.reshape(1, HID))
    agg = _sc_aggregate(srcp, dstp, h1, z128)
    h2 = _sage_ln(agg[0, :N], agg[1, :N], c0, c1, h1,
                  sage2_lW.T, sage2_lb.reshape(1, HID), sage2_rW.T,
                  ln2_g.reshape(1, HID), ln2_b.reshape(1, HID))

    agg = _sc_aggregate(srcp, dstp, h2, z128)
    emb, logp = _final(agg[0, :N], agg[1, :N], c0, c1, h2,
                       sage3_lW.T, sage3_lb.reshape(1, HID), sage3_rW.T,
                       post1_W.T, post1_b.reshape(1, HID),
                       post2_W.T, post2_b.reshape(1, OUT))
    return (emb, logp)


# speedup vs baseline: 1.8271x; 1.8271x over previous
import functools

import jax
import jax.numpy as jnp
import numpy as np
from jax import lax
from jax.experimental import pallas as pl
from jax.experimental.pallas import tpu as pltpu
from jax.experimental.pallas import tpu_sc as plsc

N = 10000
E = 320000
HID = 128
OUT = 40
CHUNK = 128
NW = 32
NCK = 80
ROWS = NW * NCK
E_PAD = ROWS * CHUNK
NACC = 10240
ZROWS = NACC // 16
BN = 400
GRID = N // BN

_f32 = jnp.float32


NCH = ZROWS // CHUNK


def _fill_stripe(stage, shared, sid):
    def body(t, carry):
        pltpu.sync_copy(stage,
                        shared.at[pl.ds(sid * ZROWS + t * CHUNK, CHUNK)])
        return carry
    lax.fori_loop(0, NCH, body, 0)


def _copy_out(shared, out_ref, stage, cid, sid):
    def body(t, carry):
        off = sid * ZROWS + t * CHUNK
        pltpu.sync_copy(shared.at[pl.ds(off, CHUNK)], stage)
        pltpu.sync_copy(stage, out_ref.at[cid, pl.ds(off, CHUNK)])
        return carry
    lax.fori_loop(0, NCH, body, 0)


def _sc_mesh():
    return plsc.VectorSubcoreMesh(core_axis_name="c", subcore_axis_name="s",
                                  num_cores=2, num_subcores=16)


def _count_kernel(dst_hbm, z128_hbm, o128_hbm, cnt_out, da, db, buf, cacc,
                  sema, semb):
    cid = lax.axis_index("c")
    sid = lax.axis_index("s")
    wid = sid * 2 + cid
    base = wid * NCK * CHUNK
    pltpu.sync_copy(z128_hbm, buf)
    _fill_stripe(buf, cacc, sid)
    pltpu.sync_copy(o128_hbm, buf)
    plsc.subcore_barrier()

    pltpu.async_copy(dst_hbm.at[pl.ds(base, CHUNK)], da, sema)
    pltpu.async_copy(dst_hbm.at[pl.ds(base + CHUNK, CHUNK)], db, semb)

    def body(kk, carry):
        k0 = 2 * kk
        pltpu.make_async_copy(dst_hbm.at[pl.ds(base, CHUNK)], da,
                              sema).wait()
        pltpu.sync_copy(buf, cacc.at[da], add=True)

        @pl.when(k0 + 2 < NCK)
        def _():
            off = base + (k0 + 2) * CHUNK
            pltpu.async_copy(dst_hbm.at[pl.ds(off, CHUNK)], da, sema)

        pltpu.make_async_copy(dst_hbm.at[pl.ds(base, CHUNK)], db,
                              semb).wait()
        pltpu.sync_copy(buf, cacc.at[db], add=True)

        @pl.when(k0 + 3 < NCK)
        def _():
            off = base + (k0 + 3) * CHUNK
            pltpu.async_copy(dst_hbm.at[pl.ds(off, CHUNK)], db, semb)

        return carry

    lax.fori_loop(0, NCK // 2, body, 0)
    plsc.subcore_barrier()
    _copy_out(cacc, cnt_out, buf, cid, sid)


def _agg_kernel(src_hbm, dst_hbm, table_hbm, z128_hbm, acc_out,
                sa, da, rows, acc, sg):
    cid = lax.axis_index("c")
    sid = lax.axis_index("s")
    wid = sid * 2 + cid
    pltpu.sync_copy(z128_hbm, rows)
    _fill_stripe(rows, acc, sid)
    plsc.subcore_barrier()

    def body(k, carry):
        off = (wid + k * NW) * CHUNK
        pltpu.sync_copy(src_hbm.at[pl.ds(off, CHUNK)], sa)
        pltpu.sync_copy(dst_hbm.at[pl.ds(off, CHUNK)], da)
        pltpu.async_copy(table_hbm.at[sa], rows, sg).wait()
        pltpu.sync_copy(rows, acc.at[da], add=True)
        return carry

    lax.fori_loop(0, NCK, body, 0)
    plsc.subcore_barrier()
    _copy_out(acc, acc_out, rows, cid, sid)


def _sc_counts(dst1d, z128, o128):
    fn = pl.kernel(
        _count_kernel,
        out_type=jax.ShapeDtypeStruct((2, NACC, HID), _f32),
        mesh=_sc_mesh(),
        scratch_types=[
            pltpu.VMEM((CHUNK,), jnp.int32),
            pltpu.VMEM((CHUNK,), jnp.int32),
            pltpu.VMEM((CHUNK, HID), _f32),
            pltpu.VMEM_SHARED((NACC, HID), _f32),
            pltpu.SemaphoreType.DMA,
            pltpu.SemaphoreType.DMA,
        ],
    )
    return fn(dst1d, z128, o128)


def _sc_aggregate(src1d, dst1d, table, z128):
    fn = pl.kernel(
        _agg_kernel,
        out_type=jax.ShapeDtypeStruct((2, NACC, HID), _f32),
        mesh=_sc_mesh(),
        scratch_types=[
            pltpu.VMEM((CHUNK,), jnp.int32),
            pltpu.VMEM((CHUNK,), jnp.int32),
            pltpu.VMEM((CHUNK, HID), _f32),
            pltpu.VMEM_SHARED((NACC, HID), _f32),
            pltpu.SemaphoreType.DMA,
        ],
    )
    return fn(src1d, dst1d, table, z128)


def _frontend_body(xf, hist, K, bfull, wsa, wsb, lsb, wna, wnb, lnb,
                   selfx, xn):
    u = jnp.maximum(xf[...] @ K[...] + bfull[...], 0.0)
    hb = jnp.maximum(hist[...], 0.0)
    selfx[...] = u @ wsa[...] + hb @ wsb[...] + lsb[...]
    xn[...] = u @ wna[...] + hb @ wnb[...] + lnb[...]


def _frontend(xf, hist, K, bfull, wsa, wsb, lsb, wna, wnb, lnb):
    row = lambda i: (i, 0)
    full = lambda i: (0, 0)
    return pl.pallas_call(
        _frontend_body,
        grid=(GRID,),
        in_specs=[
            pl.BlockSpec((BN, 400), row),
            pl.BlockSpec((BN, 16), row),
            pl.BlockSpec((400, 256), full),
            pl.BlockSpec((1, 256), full),
            pl.BlockSpec((256, HID), full),
            pl.BlockSpec((16, HID), full),
            pl.BlockSpec((1, HID), full),
            pl.BlockSpec((256, HID), full),
            pl.BlockSpec((16, HID), full),
            pl.BlockSpec((1, HID), full),
        ],
        out_specs=[pl.BlockSpec((BN, HID), row), pl.BlockSpec((BN, HID), row)],
        out_shape=[jax.ShapeDtypeStruct((N, HID), _f32),
                   jax.ShapeDtypeStruct((N, HID), _f32)],
    )(xf, hist, K, bfull, wsa, wsb, lsb, wna, wnb, lnb)


def _combine1_body(selfx, a0, a1, h0):
    h0[...] = selfx[...] + a0[...] + a1[...]


def _combine1(selfx, a0, a1):
    row = lambda i: (i, 0)
    return pl.pallas_call(
        _combine1_body,
        grid=(GRID,),
        in_specs=[pl.BlockSpec((BN, HID), row)] * 3,
        out_specs=pl.BlockSpec((BN, HID), row),
        out_shape=jax.ShapeDtypeStruct((N, HID), _f32),
    )(selfx, a0, a1)


def _sage_ln_body(a0, a1, c0, c1, h, lwt, lb, rwt, g, b, out):
    cnt = jnp.maximum(c0[...] + c1[...], 1.0)
    mean = (a0[...] + a1[...]) / cnt
    s = mean @ lwt[...] + lb[...] + h[...] @ rwt[...]
    r = jnp.maximum(s, 0.0)
    m = jnp.mean(r, axis=-1, keepdims=True)
    v = jnp.mean((r - m) ** 2, axis=-1, keepdims=True)
    out[...] = (r - m) / jnp.sqrt(v + 1e-5) * g[...] + b[...]


def _sage_ln(a0, a1, c0, c1, h, lwt, lb, rwt, g, b):
    row = lambda i: (i, 0)
    rowc = lambda i: (i, 0)
    full = lambda i: (0, 0)
    return pl.pallas_call(
        _sage_ln_body,
        grid=(GRID,),
        in_specs=[
            pl.BlockSpec((BN, HID), row),
            pl.BlockSpec((BN, HID), row),
            pl.BlockSpec((BN, 1), rowc),
            pl.BlockSpec((BN, 1), rowc),
            pl.BlockSpec((BN, HID), row),
            pl.BlockSpec((HID, HID), full),
            pl.BlockSpec((1, HID), full),
            pl.BlockSpec((HID, HID), full),
            pl.BlockSpec((1, HID), full),
            pl.BlockSpec((1, HID), full),
        ],
        out_specs=pl.BlockSpec((BN, HID), row),
        out_shape=jax.ShapeDtypeStruct((N, HID), _f32),
    )(a0, a1, c0, c1, h, lwt, lb, rwt, g, b)


def _final_body(a0, a1, c0, c1, h, lwt, lb, rwt, p1t, p1b, p2t, p2b,
                emb, logp):
    cnt = jnp.maximum(c0[...] + c1[...], 1.0)
    mean = (a0[...] + a1[...]) / cnt
    s = mean @ lwt[...] + lb[...] + h[...] @ rwt[...]
    emb[...] = s
    r = jnp.maximum(s, 0.0)
    o = r @ p1t[...] + p1b[...]
    o2 = o @ p2t[...] + p2b[...]
    mx = jnp.max(o2, axis=-1, keepdims=True)
    ex = jnp.exp(o2 - mx)
    lse = jnp.log(jnp.sum(ex, axis=-1, keepdims=True)) + mx
    logp[...] = o2 - lse


def _final(a0, a1, c0, c1, h, lwt, lb, rwt, p1t, p1b, p2t, p2b):
    row = lambda i: (i, 0)
    full = lambda i: (0, 0)
    return pl.pallas_call(
        _final_body,
        grid=(GRID,),
        in_specs=[
            pl.BlockSpec((BN, HID), row),
            pl.BlockSpec((BN, HID), row),
            pl.BlockSpec((BN, 1), row),
            pl.BlockSpec((BN, 1), row),
            pl.BlockSpec((BN, HID), row),
            pl.BlockSpec((HID, HID), full),
            pl.BlockSpec((1, HID), full),
            pl.BlockSpec((HID, HID), full),
            pl.BlockSpec((HID, HID), full),
            pl.BlockSpec((1, HID), full),
            pl.BlockSpec((HID, OUT), full),
            pl.BlockSpec((1, OUT), full),
        ],
        out_specs=[pl.BlockSpec((BN, HID), row), pl.BlockSpec((BN, OUT), row)],
        out_shape=[jax.ShapeDtypeStruct((N, HID), _f32),
                   jax.ShapeDtypeStruct((N, OUT), _f32)],
    )(a0, a1, c0, c1, h, lwt, lb, rwt, p1t, p1b, p2t, p2b)


_o, _c, _di, _dj, _i, _j = np.meshgrid(
    np.arange(4), np.arange(4), np.arange(3), np.arange(3),
    np.arange(8), np.arange(8), indexing="ij")
_RIN = (_c * 100 + (_i + _di) * 10 + (_j + _dj)).reshape(-1)
_ROUT = (_o * 64 + _i * 8 + _j).reshape(-1)


def _conv_as_matmul(conv_W):
    vals = jnp.broadcast_to(conv_W[:, :, :, :, None, None],
                            (4, 4, 3, 3, 8, 8)).reshape(-1)
    return jnp.zeros((400, 256), _f32).at[_RIN, _ROUT].set(vals)


def kernel(x, histories_preds, edge_index, batch, conv_self_W, conv_self_b,
           lin_self_W, lin_self_b, lin_W, lin_b,
           sage1_lW, sage1_lb, sage1_rW, sage2_lW, sage2_lb, sage2_rW,
           sage3_lW, sage3_lb, sage3_rW,
           ln1_g, ln1_b, ln2_g, ln2_b, post1_W, post1_b, post2_W, post2_b):
    src = edge_index[0]
    dst = edge_index[1]
    dstm = jnp.where(src == dst, N, dst)
    pad = E_PAD - E
    padi = jnp.arange(pad, dtype=jnp.int32)
    trash = N + padi % (NACC - N)
    srcp = jnp.concatenate([src, padi % N])
    dstp = jnp.concatenate([dst, trash])
    dstmp = jnp.concatenate([dstm, trash])

    xf = x.reshape(N, 400)
    K = _conv_as_matmul(conv_self_W)
    bfull = jnp.repeat(conv_self_b, 64).reshape(1, 256)
    wsa = lin_self_W[:, :256].T
    wsb = lin_self_W[:, 256:].T
    wna = lin_W[:, :256].T
    wnb = lin_W[:, 256:].T

    z128 = jnp.zeros((CHUNK, HID), _f32)
    o128 = jnp.ones((CHUNK, HID), _f32)

    selfx, xn = _frontend(xf, histories_preds, K, bfull,
                          wsa, wsb, lin_self_b.reshape(1, HID),
                          wna, wnb, lin_b.reshape(1, HID))

    cnt = _sc_counts(dstp, z128, o128)
    acc1 = _sc_aggregate(srcp, dstmp, xn, z128)
    c0 = cnt[0, :N, 0].reshape(N, 1)
    c1 = cnt[1, :N, 0].reshape(N, 1)
    h0 = _combine1(selfx, acc1[0, :N], acc1[1, :N])

    agg = _sc_aggregate(srcp, dstp, h0, z128)
    h1 = _sage_ln(agg[0, :N], agg[1, :N], c0, c1, h0,
                  sage1_lW.T, sage1_lb.reshape(1, HID), sage1_rW.T,
                  ln1_g.reshape(1, HID), ln1_b.reshape(1, HID))
    agg = _sc_aggregate(srcp, dstp, h1, z128)
    h2 = _sage_ln(agg[0, :N], agg[1, :N], c0, c1, h1,
                  sage2_lW.T, sage2_lb.reshape(1, HID), sage2_rW.T,
                  ln2_g.reshape(1, HID), ln2_b.reshape(1, HID))

    agg = _sc_aggregate(srcp, dstp, h2, z128)
    emb, logp = _final(agg[0, :N], agg[1, :N], c0, c1, h2,
                       sage3_lW.T, sage3_lb.reshape(1, HID), sage3_rW.T,
                       post1_W.T, post1_b.reshape(1, HID),
                       post2_W.T, post2_b.reshape(1, OUT))
    return (emb, logp)


# speedup vs baseline: 2.4178x; 1.3233x over previous
import functools

import jax
import jax.numpy as jnp
import numpy as np
from jax import lax
from jax.experimental import pallas as pl
from jax.experimental.pallas import tpu as pltpu
from jax.experimental.pallas import tpu_sc as plsc

N = 10000
E = 320000
HID = 128
OUT = 40
CHUNK = 64
NW = 32
NCK = 158
ROWS = NW * NCK
E_PAD = ROWS * CHUNK
NACC = 10240
ZROWS = NACC // 16
BN = 400
GRID = N // BN

_f32 = jnp.float32


NCH = ZROWS // CHUNK


def _fill_stripe(stage, shared, sid):
    def body(t, carry):
        pltpu.sync_copy(stage,
                        shared.at[pl.ds(sid * ZROWS + t * CHUNK, CHUNK)])
        return carry
    lax.fori_loop(0, NCH, body, 0)


def _copy_out(shared, out_ref, stage, cid, sid):
    def body(t, carry):
        off = sid * ZROWS + t * CHUNK
        pltpu.sync_copy(shared.at[pl.ds(off, CHUNK)], stage)
        pltpu.sync_copy(stage, out_ref.at[cid, pl.ds(off, CHUNK)])
        return carry
    lax.fori_loop(0, NCH, body, 0)


def _sc_mesh():
    return plsc.VectorSubcoreMesh(core_axis_name="c", subcore_axis_name="s",
                                  num_cores=2, num_subcores=16)


def _count_kernel(dst_hbm, z128_hbm, o128_hbm, cnt_out, da, db, buf, cacc,
                  sema, semb):
    cid = lax.axis_index("c")
    sid = lax.axis_index("s")
    wid = sid * 2 + cid
    base = wid * NCK * CHUNK
    pltpu.sync_copy(z128_hbm, buf)
    _fill_stripe(buf, cacc, sid)
    pltpu.sync_copy(o128_hbm, buf)
    plsc.subcore_barrier()

    pltpu.async_copy(dst_hbm.at[pl.ds(base, CHUNK)], da, sema)
    pltpu.async_copy(dst_hbm.at[pl.ds(base + CHUNK, CHUNK)], db, semb)

    def body(kk, carry):
        k0 = 2 * kk
        pltpu.make_async_copy(dst_hbm.at[pl.ds(base, CHUNK)], da,
                              sema).wait()
        pltpu.sync_copy(buf, cacc.at[da], add=True)

        @pl.when(k0 + 2 < NCK)
        def _():
            off = base + (k0 + 2) * CHUNK
            pltpu.async_copy(dst_hbm.at[pl.ds(off, CHUNK)], da, sema)

        pltpu.make_async_copy(dst_hbm.at[pl.ds(base, CHUNK)], db,
                              semb).wait()
        pltpu.sync_copy(buf, cacc.at[db], add=True)

        @pl.when(k0 + 3 < NCK)
        def _():
            off = base + (k0 + 3) * CHUNK
            pltpu.async_copy(dst_hbm.at[pl.ds(off, CHUNK)], db, semb)

        return carry

    lax.fori_loop(0, NCK // 2, body, 0)
    plsc.subcore_barrier()
    _copy_out(cacc, cnt_out, buf, cid, sid)


def _agg_kernel(src_hbm, dst_hbm, table_hbm, z128_hbm, acc_out,
                sa, da, sb, db, rowsa, rowsb, acc,
                semga, semgb, semia, semib):
    cid = lax.axis_index("c")
    sid = lax.axis_index("s")
    wid = sid * 2 + cid
    base = wid * NCK * CHUNK
    pltpu.sync_copy(z128_hbm, rowsa)
    _fill_stripe(rowsa, acc, sid)
    plsc.subcore_barrier()

    pltpu.sync_copy(src_hbm.at[pl.ds(base, CHUNK)], sa)
    pltpu.sync_copy(dst_hbm.at[pl.ds(base, CHUNK)], da)
    pltpu.async_copy(table_hbm.at[sa], rowsa, semga)
    pltpu.async_copy(src_hbm.at[pl.ds(base + CHUNK, CHUNK)], sb, semib)
    pltpu.async_copy(dst_hbm.at[pl.ds(base + CHUNK, CHUNK)], db, semib)

    def body(kk, carry):
        k0 = 2 * kk
        pltpu.make_async_copy(src_hbm.at[pl.ds(base, CHUNK)], sb,
                              semib).wait()
        pltpu.make_async_copy(dst_hbm.at[pl.ds(base, CHUNK)], db,
                              semib).wait()
        pltpu.async_copy(table_hbm.at[sb], rowsb, semgb)
        pltpu.make_async_copy(table_hbm.at[sa], rowsa, semga).wait()
        pltpu.sync_copy(rowsa, acc.at[da], add=True)

        @pl.when(k0 + 2 < NCK)
        def _():
            off = base + (k0 + 2) * CHUNK
            pltpu.async_copy(src_hbm.at[pl.ds(off, CHUNK)], sa, semia)
            pltpu.async_copy(dst_hbm.at[pl.ds(off, CHUNK)], da, semia)

        pltpu.make_async_copy(table_hbm.at[sb], rowsb, semgb).wait()
        pltpu.sync_copy(rowsb, acc.at[db], add=True)

        @pl.when(k0 + 2 < NCK)
        def _():
            pltpu.make_async_copy(src_hbm.at[pl.ds(base, CHUNK)], sa,
                                  semia).wait()
            pltpu.make_async_copy(dst_hbm.at[pl.ds(base, CHUNK)], da,
                                  semia).wait()
            pltpu.async_copy(table_hbm.at[sa], rowsa, semga)

        @pl.when(k0 + 3 < NCK)
        def _():
            off = base + (k0 + 3) * CHUNK
            pltpu.async_copy(src_hbm.at[pl.ds(off, CHUNK)], sb, semib)
            pltpu.async_copy(dst_hbm.at[pl.ds(off, CHUNK)], db, semib)

        return carry

    lax.fori_loop(0, NCK // 2, body, 0)
    plsc.subcore_barrier()
    _copy_out(acc, acc_out, rowsa, cid, sid)


def _sc_counts(dst1d, z128, o128):
    fn = pl.kernel(
        _count_kernel,
        out_type=jax.ShapeDtypeStruct((2, NACC, HID), _f32),
        mesh=_sc_mesh(),
        scratch_types=[
            pltpu.VMEM((CHUNK,), jnp.int32),
            pltpu.VMEM((CHUNK,), jnp.int32),
            pltpu.VMEM((CHUNK, HID), _f32),
            pltpu.VMEM_SHARED((NACC, HID), _f32),
            pltpu.SemaphoreType.DMA,
            pltpu.SemaphoreType.DMA,
        ],
    )
    return fn(dst1d, z128, o128)


def _sc_aggregate(src1d, dst1d, table, z128):
    fn = pl.kernel(
        _agg_kernel,
        out_type=jax.ShapeDtypeStruct((2, NACC, HID), _f32),
        mesh=_sc_mesh(),
        scratch_types=[
            pltpu.VMEM((CHUNK,), jnp.int32),
            pltpu.VMEM((CHUNK,), jnp.int32),
            pltpu.VMEM((CHUNK,), jnp.int32),
            pltpu.VMEM((CHUNK,), jnp.int32),
            pltpu.VMEM((CHUNK, HID), _f32),
            pltpu.VMEM((CHUNK, HID), _f32),
            pltpu.VMEM_SHARED((NACC, HID), _f32),
            pltpu.SemaphoreType.DMA,
            pltpu.SemaphoreType.DMA,
            pltpu.SemaphoreType.DMA,
            pltpu.SemaphoreType.DMA,
        ],
    )
    return fn(src1d, dst1d, table, z128)


def _frontend_body(xf, hist, K, bfull, wsa, wsb, lsb, wna, wnb, lnb,
                   selfx, xn):
    u = jnp.maximum(xf[...] @ K[...] + bfull[...], 0.0)
    hb = jnp.maximum(hist[...], 0.0)
    selfx[...] = u @ wsa[...] + hb @ wsb[...] + lsb[...]
    xn[...] = u @ wna[...] + hb @ wnb[...] + lnb[...]


def _frontend(xf, hist, K, bfull, wsa, wsb, lsb, wna, wnb, lnb):
    row = lambda i: (i, 0)
    full = lambda i: (0, 0)
    return pl.pallas_call(
        _frontend_body,
        grid=(GRID,),
        in_specs=[
            pl.BlockSpec((BN, 400), row),
            pl.BlockSpec((BN, 16), row),
            pl.BlockSpec((400, 256), full),
            pl.BlockSpec((1, 256), full),
            pl.BlockSpec((256, HID), full),
            pl.BlockSpec((16, HID), full),
            pl.BlockSpec((1, HID), full),
            pl.BlockSpec((256, HID), full),
            pl.BlockSpec((16, HID), full),
            pl.BlockSpec((1, HID), full),
        ],
        out_specs=[pl.BlockSpec((BN, HID), row), pl.BlockSpec((BN, HID), row)],
        out_shape=[jax.ShapeDtypeStruct((N, HID), _f32),
                   jax.ShapeDtypeStruct((N, HID), _f32)],
    )(xf, hist, K, bfull, wsa, wsb, lsb, wna, wnb, lnb)


def _combine1_body(selfx, a0, a1, h0):
    h0[...] = selfx[...] + a0[...] + a1[...]


def _combine1(selfx, a0, a1):
    row = lambda i: (i, 0)
    return pl.pallas_call(
        _combine1_body,
        grid=(GRID,),
        in_specs=[pl.BlockSpec((BN, HID), row)] * 3,
        out_specs=pl.BlockSpec((BN, HID), row),
        out_shape=jax.ShapeDtypeStruct((N, HID), _f32),
    )(selfx, a0, a1)


def _sage_ln_body(a0, a1, c0, c1, h, lwt, lb, rwt, g, b, out):
    cnt = jnp.maximum(c0[...] + c1[...], 1.0)
    mean = (a0[...] + a1[...]) / cnt
    s = mean @ lwt[...] + lb[...] + h[...] @ rwt[...]
    r = jnp.maximum(s, 0.0)
    m = jnp.mean(r, axis=-1, keepdims=True)
    v = jnp.mean((r - m) ** 2, axis=-1, keepdims=True)
    out[...] = (r - m) / jnp.sqrt(v + 1e-5) * g[...] + b[...]


def _sage_ln(a0, a1, c0, c1, h, lwt, lb, rwt, g, b):
    row = lambda i: (i, 0)
    rowc = lambda i: (i, 0)
    full = lambda i: (0, 0)
    return pl.pallas_call(
        _sage_ln_body,
        grid=(GRID,),
        in_specs=[
            pl.BlockSpec((BN, HID), row),
            pl.BlockSpec((BN, HID), row),
            pl.BlockSpec((BN, 1), rowc),
            pl.BlockSpec((BN, 1), rowc),
            pl.BlockSpec((BN, HID), row),
            pl.BlockSpec((HID, HID), full),
            pl.BlockSpec((1, HID), full),
            pl.BlockSpec((HID, HID), full),
            pl.BlockSpec((1, HID), full),
            pl.BlockSpec((1, HID), full),
        ],
        out_specs=pl.BlockSpec((BN, HID), row),
        out_shape=jax.ShapeDtypeStruct((N, HID), _f32),
    )(a0, a1, c0, c1, h, lwt, lb, rwt, g, b)


def _final_body(a0, a1, c0, c1, h, lwt, lb, rwt, p1t, p1b, p2t, p2b,
                emb, logp):
    cnt = jnp.maximum(c0[...] + c1[...], 1.0)
    mean = (a0[...] + a1[...]) / cnt
    s = mean @ lwt[...] + lb[...] + h[...] @ rwt[...]
    emb[...] = s
    r = jnp.maximum(s, 0.0)
    o = r @ p1t[...] + p1b[...]
    o2 = o @ p2t[...] + p2b[...]
    mx = jnp.max(o2, axis=-1, keepdims=True)
    ex = jnp.exp(o2 - mx)
    lse = jnp.log(jnp.sum(ex, axis=-1, keepdims=True)) + mx
    logp[...] = o2 - lse


def _final(a0, a1, c0, c1, h, lwt, lb, rwt, p1t, p1b, p2t, p2b):
    row = lambda i: (i, 0)
    full = lambda i: (0, 0)
    return pl.pallas_call(
        _final_body,
        grid=(GRID,),
        in_specs=[
            pl.BlockSpec((BN, HID), row),
            pl.BlockSpec((BN, HID), row),
            pl.BlockSpec((BN, 1), row),
            pl.BlockSpec((BN, 1), row),
            pl.BlockSpec((BN, HID), row),
            pl.BlockSpec((HID, HID), full),
            pl.BlockSpec((1, HID), full),
            pl.BlockSpec((HID, HID), full),
            pl.BlockSpec((HID, HID), full),
            pl.BlockSpec((1, HID), full),
            pl.BlockSpec((HID, OUT), full),
            pl.BlockSpec((1, OUT), full),
        ],
        out_specs=[pl.BlockSpec((BN, HID), row), pl.BlockSpec((BN, OUT), row)],
        out_shape=[jax.ShapeDtypeStruct((N, HID), _f32),
                   jax.ShapeDtypeStruct((N, OUT), _f32)],
    )(a0, a1, c0, c1, h, lwt, lb, rwt, p1t, p1b, p2t, p2b)


_o, _c, _di, _dj, _i, _j = np.meshgrid(
    np.arange(4), np.arange(4), np.arange(3), np.arange(3),
    np.arange(8), np.arange(8), indexing="ij")
_RIN = (_c * 100 + (_i + _di) * 10 + (_j + _dj)).reshape(-1)
_ROUT = (_o * 64 + _i * 8 + _j).reshape(-1)


def _conv_as_matmul(conv_W):
    vals = jnp.broadcast_to(conv_W[:, :, :, :, None, None],
                            (4, 4, 3, 3, 8, 8)).reshape(-1)
    return jnp.zeros((400, 256), _f32).at[_RIN, _ROUT].set(vals)


def kernel(x, histories_preds, edge_index, batch, conv_self_W, conv_self_b,
           lin_self_W, lin_self_b, lin_W, lin_b,
           sage1_lW, sage1_lb, sage1_rW, sage2_lW, sage2_lb, sage2_rW,
           sage3_lW, sage3_lb, sage3_rW,
           ln1_g, ln1_b, ln2_g, ln2_b, post1_W, post1_b, post2_W, post2_b):
    src = edge_index[0]
    dst = edge_index[1]
    dstm = jnp.where(src == dst, N, dst)
    pad = E_PAD - E
    padi = jnp.arange(pad, dtype=jnp.int32)
    trash = N + padi % (NACC - N)
    srcp = jnp.concatenate([src, padi % N])
    dstp = jnp.concatenate([dst, trash])
    dstmp = jnp.concatenate([dstm, trash])

    xf = x.reshape(N, 400)
    K = _conv_as_matmul(conv_self_W)
    bfull = jnp.repeat(conv_self_b, 64).reshape(1, 256)
    wsa = lin_self_W[:, :256].T
    wsb = lin_self_W[:, 256:].T
    wna = lin_W[:, :256].T
    wnb = lin_W[:, 256:].T

    z128 = jnp.zeros((CHUNK, HID), _f32)
    o128 = jnp.ones((CHUNK, HID), _f32)

    selfx, xn = _frontend(xf, histories_preds, K, bfull,
                          wsa, wsb, lin_self_b.reshape(1, HID),
                          wna, wnb, lin_b.reshape(1, HID))

    cnt = _sc_counts(dstp, z128, o128)
    acc1 = _sc_aggregate(srcp, dstmp, xn, z128)
    c0 = cnt[0, :N, 0].reshape(N, 1)
    c1 = cnt[1, :N, 0].reshape(N, 1)
    h0 = _combine1(selfx, acc1[0, :N], acc1[1, :N])

    agg = _sc_aggregate(srcp, dstp, h0, z128)
    h1 = _sage_ln(agg[0, :N], agg[1, :N], c0, c1, h0,
                  sage1_lW.T, sage1_lb.reshape(1, HID), sage1_rW.T,
                  ln1_g.reshape(1, HID), ln1_b.reshape(1, HID))
    agg = _sc_aggregate(srcp, dstp, h1, z128)
    h2 = _sage_ln(agg[0, :N], agg[1, :N], c0, c1, h1,
                  sage2_lW.T, sage2_lb.reshape(1, HID), sage2_rW.T,
                  ln2_g.reshape(1, HID), ln2_b.reshape(1, HID))

    agg = _sc_aggregate(srcp, dstp, h2, z128)
    emb, logp = _final(agg[0, :N], agg[1, :N], c0, c1, h2,
                       sage3_lW.T, sage3_lb.reshape(1, HID), sage3_rW.T,
                       post1_W.T, post1_b.reshape(1, HID),
                       post2_W.T, post2_b.reshape(1, OUT))
    return (emb, logp)


# speedup vs baseline: 2.7597x; 1.1414x over previous
import functools

import jax
import jax.numpy as jnp
import numpy as np
from jax import lax
from jax.experimental import pallas as pl
from jax.experimental.pallas import tpu as pltpu
from jax.experimental.pallas import tpu_sc as plsc

N = 10000
E = 320000
HID = 128
OUT = 40
CHUNK = 128
NW = 32
NCK = 80
ROWS = NW * NCK
E_PAD = ROWS * CHUNK
NACC = 10240
ZROWS = NACC // 16
BN = 400
GRID = N // BN

_f32 = jnp.float32


NCH = ZROWS // CHUNK


def _fill_stripe(stage, shared, sid):
    def body(t, carry):
        pltpu.sync_copy(stage,
                        shared.at[pl.ds(sid * ZROWS + t * CHUNK, CHUNK)])
        return carry
    lax.fori_loop(0, NCH, body, 0)


def _copy_out(shared, out_ref, stage, cid, sid):
    def body(t, carry):
        off = sid * ZROWS + t * CHUNK
        pltpu.sync_copy(shared.at[pl.ds(off, CHUNK)], stage)
        pltpu.sync_copy(stage, out_ref.at[cid, pl.ds(off, CHUNK)])
        return carry
    lax.fori_loop(0, NCH, body, 0)


def _sc_mesh():
    return plsc.VectorSubcoreMesh(core_axis_name="c", subcore_axis_name="s",
                                  num_cores=2, num_subcores=16)


def _count_kernel(dst_hbm, z128_hbm, o128_hbm, cnt_out, da, db, buf, cacc,
                  sema, semb):
    cid = lax.axis_index("c")
    sid = lax.axis_index("s")
    wid = sid * 2 + cid
    base = wid * NCK * CHUNK
    pltpu.sync_copy(z128_hbm, buf)
    _fill_stripe(buf, cacc, sid)
    pltpu.sync_copy(o128_hbm, buf)
    plsc.subcore_barrier()

    pltpu.async_copy(dst_hbm.at[pl.ds(base, CHUNK)], da, sema)
    pltpu.async_copy(dst_hbm.at[pl.ds(base + CHUNK, CHUNK)], db, semb)

    def body(kk, carry):
        k0 = 2 * kk
        pltpu.make_async_copy(dst_hbm.at[pl.ds(base, CHUNK)], da,
                              sema).wait()
        pltpu.sync_copy(buf, cacc.at[da], add=True)

        @pl.when(k0 + 2 < NCK)
        def _():
            off = base + (k0 + 2) * CHUNK
            pltpu.async_copy(dst_hbm.at[pl.ds(off, CHUNK)], da, sema)

        pltpu.make_async_copy(dst_hbm.at[pl.ds(base, CHUNK)], db,
                              semb).wait()
        pltpu.sync_copy(buf, cacc.at[db], add=True)

        @pl.when(k0 + 3 < NCK)
        def _():
            off = base + (k0 + 3) * CHUNK
            pltpu.async_copy(dst_hbm.at[pl.ds(off, CHUNK)], db, semb)

        return carry

    lax.fori_loop(0, NCK // 2, body, 0)
    plsc.subcore_barrier()
    _copy_out(cacc, cnt_out, buf, cid, sid)


def _agg_kernel(src_hbm, dst_hbm, table_hbm, z128_hbm, acc_out,
                sa, da, sb, db, rowsa, rowsb, acc,
                semga, semgb, semia, semib):
    cid = lax.axis_index("c")
    sid = lax.axis_index("s")
    wid = sid * 2 + cid
    base = wid * NCK * CHUNK
    pltpu.sync_copy(z128_hbm, rowsa)
    _fill_stripe(rowsa, acc, sid)
    plsc.subcore_barrier()

    pltpu.sync_copy(src_hbm.at[pl.ds(base, CHUNK)], sa)
    pltpu.sync_copy(dst_hbm.at[pl.ds(base, CHUNK)], da)
    pltpu.async_copy(table_hbm.at[sa], rowsa, semga)
    pltpu.async_copy(src_hbm.at[pl.ds(base + CHUNK, CHUNK)], sb, semib)
    pltpu.async_copy(dst_hbm.at[pl.ds(base + CHUNK, CHUNK)], db, semib)

    def body(kk, carry):
        k0 = 2 * kk
        pltpu.make_async_copy(src_hbm.at[pl.ds(base, CHUNK)], sb,
                              semib).wait()
        pltpu.make_async_copy(dst_hbm.at[pl.ds(base, CHUNK)], db,
                              semib).wait()
        pltpu.async_copy(table_hbm.at[sb], rowsb, semgb)
        pltpu.make_async_copy(table_hbm.at[sa], rowsa, semga).wait()
        pltpu.sync_copy(rowsa, acc.at[da], add=True)

        @pl.when(k0 + 2 < NCK)
        def _():
            off = base + (k0 + 2) * CHUNK
            pltpu.async_copy(src_hbm.at[pl.ds(off, CHUNK)], sa, semia)
            pltpu.async_copy(dst_hbm.at[pl.ds(off, CHUNK)], da, semia)

        pltpu.make_async_copy(table_hbm.at[sb], rowsb, semgb).wait()
        pltpu.sync_copy(rowsb, acc.at[db], add=True)

        @pl.when(k0 + 2 < NCK)
        def _():
            pltpu.make_async_copy(src_hbm.at[pl.ds(base, CHUNK)], sa,
                                  semia).wait()
            pltpu.make_async_copy(dst_hbm.at[pl.ds(base, CHUNK)], da,
                                  semia).wait()
            pltpu.async_copy(table_hbm.at[sa], rowsa, semga)

        @pl.when(k0 + 3 < NCK)
        def _():
            off = base + (k0 + 3) * CHUNK
            pltpu.async_copy(src_hbm.at[pl.ds(off, CHUNK)], sb, semib)
            pltpu.async_copy(dst_hbm.at[pl.ds(off, CHUNK)], db, semib)

        return carry

    lax.fori_loop(0, NCK // 2, body, 0)
    plsc.subcore_barrier()
    _copy_out(acc, acc_out, rowsa, cid, sid)


def _sc_counts(dst1d, z128, o128):
    fn = pl.kernel(
        _count_kernel,
        out_type=jax.ShapeDtypeStruct((2, NACC, HID), _f32),
        mesh=_sc_mesh(),
        scratch_types=[
            pltpu.VMEM((CHUNK,), jnp.int32),
            pltpu.VMEM((CHUNK,), jnp.int32),
            pltpu.VMEM((CHUNK, HID), _f32),
            pltpu.VMEM_SHARED((NACC, HID), _f32),
            pltpu.SemaphoreType.DMA,
            pltpu.SemaphoreType.DMA,
        ],
    )
    return fn(dst1d, z128, o128)


def _sc_aggregate(src1d, dst1d, table, z128):
    fn = pl.kernel(
        _agg_kernel,
        out_type=jax.ShapeDtypeStruct((2, NACC, HID), _f32),
        mesh=_sc_mesh(),
        scratch_types=[
            pltpu.VMEM((CHUNK,), jnp.int32),
            pltpu.VMEM((CHUNK,), jnp.int32),
            pltpu.VMEM((CHUNK,), jnp.int32),
            pltpu.VMEM((CHUNK,), jnp.int32),
            pltpu.VMEM((CHUNK, HID), _f32),
            pltpu.VMEM((CHUNK, HID), _f32),
            pltpu.VMEM_SHARED((NACC, HID), _f32),
            pltpu.SemaphoreType.DMA,
            pltpu.SemaphoreType.DMA,
            pltpu.SemaphoreType.DMA,
            pltpu.SemaphoreType.DMA,
        ],
    )
    return fn(src1d, dst1d, table, z128)


def _frontend_body(xf, hist, K, bfull, wsa, wsb, lsb, wna, wnb, lnb,
                   selfx, xn):
    u = jnp.maximum(xf[...] @ K[...] + bfull[...], 0.0)
    hb = jnp.maximum(hist[...], 0.0)
    selfx[...] = u @ wsa[...] + hb @ wsb[...] + lsb[...]
    xn[...] = u @ wna[...] + hb @ wnb[...] + lnb[...]


def _frontend(xf, hist, K, bfull, wsa, wsb, lsb, wna, wnb, lnb):
    row = lambda i: (i, 0)
    full = lambda i: (0, 0)
    return pl.pallas_call(
        _frontend_body,
        grid=(GRID,),
        in_specs=[
            pl.BlockSpec((BN, 400), row),
            pl.BlockSpec((BN, 16), row),
            pl.BlockSpec((400, 256), full),
            pl.BlockSpec((1, 256), full),
            pl.BlockSpec((256, HID), full),
            pl.BlockSpec((16, HID), full),
            pl.BlockSpec((1, HID), full),
            pl.BlockSpec((256, HID), full),
            pl.BlockSpec((16, HID), full),
            pl.BlockSpec((1, HID), full),
        ],
        out_specs=[pl.BlockSpec((BN, HID), row), pl.BlockSpec((BN, HID), row)],
        out_shape=[jax.ShapeDtypeStruct((N, HID), _f32),
                   jax.ShapeDtypeStruct((N, HID), _f32)],
    )(xf, hist, K, bfull, wsa, wsb, lsb, wna, wnb, lnb)


def _combine1_body(selfx, a0, a1, h0):
    h0[...] = selfx[...] + a0[...] + a1[...]


def _combine1(selfx, a0, a1):
    row = lambda i: (i, 0)
    return pl.pallas_call(
        _combine1_body,
        grid=(GRID,),
        in_specs=[pl.BlockSpec((BN, HID), row)] * 3,
        out_specs=pl.BlockSpec((BN, HID), row),
        out_shape=jax.ShapeDtypeStruct((N, HID), _f32),
    )(selfx, a0, a1)


def _sage_ln_body(a0, a1, c0, c1, h, lwt, lb, rwt, g, b, out):
    cnt = jnp.maximum(c0[...] + c1[...], 1.0)
    mean = (a0[...] + a1[...]) / cnt
    s = mean @ lwt[...] + lb[...] + h[...] @ rwt[...]
    r = jnp.maximum(s, 0.0)
    m = jnp.mean(r, axis=-1, keepdims=True)
    v = jnp.mean((r - m) ** 2, axis=-1, keepdims=True)
    out[...] = (r - m) / jnp.sqrt(v + 1e-5) * g[...] + b[...]


def _sage_ln(a0, a1, c0, c1, h, lwt, lb, rwt, g, b):
    row = lambda i: (i, 0)
    rowc = lambda i: (i, 0)
    full = lambda i: (0, 0)
    return pl.pallas_call(
        _sage_ln_body,
        grid=(GRID,),
        in_specs=[
            pl.BlockSpec((BN, HID), row),
            pl.BlockSpec((BN, HID), row),
            pl.BlockSpec((BN, 1), rowc),
            pl.BlockSpec((BN, 1), rowc),
            pl.BlockSpec((BN, HID), row),
            pl.BlockSpec((HID, HID), full),
            pl.BlockSpec((1, HID), full),
            pl.BlockSpec((HID, HID), full),
            pl.BlockSpec((1, HID), full),
            pl.BlockSpec((1, HID), full),
        ],
        out_specs=pl.BlockSpec((BN, HID), row),
        out_shape=jax.ShapeDtypeStruct((N, HID), _f32),
    )(a0, a1, c0, c1, h, lwt, lb, rwt, g, b)


def _final_body(a0, a1, c0, c1, h, lwt, lb, rwt, p1t, p1b, p2t, p2b,
                emb, logp):
    cnt = jnp.maximum(c0[...] + c1[...], 1.0)
    mean = (a0[...] + a1[...]) / cnt
    s = mean @ lwt[...] + lb[...] + h[...] @ rwt[...]
    emb[...] = s
    r = jnp.maximum(s, 0.0)
    o = r @ p1t[...] + p1b[...]
    o2 = o @ p2t[...] + p2b[...]
    mx = jnp.max(o2, axis=-1, keepdims=True)
    ex = jnp.exp(o2 - mx)
    lse = jnp.log(jnp.sum(ex, axis=-1, keepdims=True)) + mx
    logp[...] = o2 - lse


def _final(a0, a1, c0, c1, h, lwt, lb, rwt, p1t, p1b, p2t, p2b):
    row = lambda i: (i, 0)
    full = lambda i: (0, 0)
    return pl.pallas_call(
        _final_body,
        grid=(GRID,),
        in_specs=[
            pl.BlockSpec((BN, HID), row),
            pl.BlockSpec((BN, HID), row),
            pl.BlockSpec((BN, 1), row),
            pl.BlockSpec((BN, 1), row),
            pl.BlockSpec((BN, HID), row),
            pl.BlockSpec((HID, HID), full),
            pl.BlockSpec((1, HID), full),
            pl.BlockSpec((HID, HID), full),
            pl.BlockSpec((HID, HID), full),
            pl.BlockSpec((1, HID), full),
            pl.BlockSpec((HID, OUT), full),
            pl.BlockSpec((1, OUT), full),
        ],
        out_specs=[pl.BlockSpec((BN, HID), row), pl.BlockSpec((BN, OUT), row)],
        out_shape=[jax.ShapeDtypeStruct((N, HID), _f32),
                   jax.ShapeDtypeStruct((N, OUT), _f32)],
    )(a0, a1, c0, c1, h, lwt, lb, rwt, p1t, p1b, p2t, p2b)


_o, _c, _di, _dj, _i, _j = np.meshgrid(
    np.arange(4), np.arange(4), np.arange(3), np.arange(3),
    np.arange(8), np.arange(8), indexing="ij")
_RIN = (_c * 100 + (_i + _di) * 10 + (_j + _dj)).reshape(-1)
_ROUT = (_o * 64 + _i * 8 + _j).reshape(-1)


def _conv_as_matmul(conv_W):
    vals = jnp.broadcast_to(conv_W[:, :, :, :, None, None],
                            (4, 4, 3, 3, 8, 8)).reshape(-1)
    return jnp.zeros((400, 256), _f32).at[_RIN, _ROUT].set(vals)


def kernel(x, histories_preds, edge_index, batch, conv_self_W, conv_self_b,
           lin_self_W, lin_self_b, lin_W, lin_b,
           sage1_lW, sage1_lb, sage1_rW, sage2_lW, sage2_lb, sage2_rW,
           sage3_lW, sage3_lb, sage3_rW,
           ln1_g, ln1_b, ln2_g, ln2_b, post1_W, post1_b, post2_W, post2_b):
    src = edge_index[0]
    dst = edge_index[1]
    dstm = jnp.where(src == dst, N, dst)
    pad = E_PAD - E
    padi = jnp.arange(pad, dtype=jnp.int32)
    trash = N + padi % (NACC - N)
    srcp = jnp.concatenate([src, padi % N])
    dstp = jnp.concatenate([dst, trash])
    dstmp = jnp.concatenate([dstm, trash])

    xf = x.reshape(N, 400)
    K = _conv_as_matmul(conv_self_W)
    bfull = jnp.repeat(conv_self_b, 64).reshape(1, 256)
    wsa = lin_self_W[:, :256].T
    wsb = lin_self_W[:, 256:].T
    wna = lin_W[:, :256].T
    wnb = lin_W[:, 256:].T

    z128 = jnp.zeros((CHUNK, HID), _f32)
    o128 = jnp.ones((CHUNK, HID), _f32)

    selfx, xn = _frontend(xf, histories_preds, K, bfull,
                          wsa, wsb, lin_self_b.reshape(1, HID),
                          wna, wnb, lin_b.reshape(1, HID))

    cnt = _sc_counts(dstp, z128, o128)
    acc1 = _sc_aggregate(srcp, dstmp, xn, z128)
    c0 = cnt[0, :N, 0].reshape(N, 1)
    c1 = cnt[1, :N, 0].reshape(N, 1)
    h0 = _combine1(selfx, acc1[0, :N], acc1[1, :N])

    agg = _sc_aggregate(srcp, dstp, h0, z128)
    h1 = _sage_ln(agg[0, :N], agg[1, :N], c0, c1, h0,
                  sage1_lW.T, sage1_lb.reshape(1, HID), sage1_rW.T,
                  ln1_g.reshape(1, HID), ln1_b.reshape(1, HID))
    agg = _sc_aggregate(srcp, dstp, h1, z128)
    h2 = _sage_ln(agg[0, :N], agg[1, :N], c0, c1, h1,
                  sage2_lW.T, sage2_lb.reshape(1, HID), sage2_rW.T,
                  ln2_g.reshape(1, HID), ln2_b.reshape(1, HID))

    agg = _sc_aggregate(srcp, dstp, h2, z128)
    emb, logp = _final(agg[0, :N], agg[1, :N], c0, c1, h2,
                       sage3_lW.T, sage3_lb.reshape(1, HID), sage3_rW.T,
                       post1_W.T, post1_b.reshape(1, HID),
                       post2_W.T, post2_b.reshape(1, OUT))
    return (emb, logp)
